# Initial kernel scaffold; baseline (speedup 1.0000x reference)
import jax
import jax.numpy as jnp
from jax.experimental import pallas as pl


def kernel(inputs, table):
    raise NotImplementedError("write your pallas kernel here")


# speedup vs baseline: 1.8295x; 1.8295x over previous
import functools

import jax
import jax.numpy as jnp
from jax import lax
from jax.experimental import pallas as pl
from jax.experimental.pallas import tpu as pltpu
from jax.experimental.pallas import tpu_sc as plsc

B = 16384
L = 50
EMBED = 64
TOTAL = B * L

_info = plsc.get_sparse_core_info()
NC = _info.num_cores
NS = _info.num_subcores
NW = NC * NS
B_PER_W = TOTAL // NW
CHUNK = 512
N_CHUNKS = B_PER_W // CHUNK

_mesh = plsc.VectorSubcoreMesh(core_axis_name="c", subcore_axis_name="s")


@functools.partial(
    pl.kernel,
    out_type=jax.ShapeDtypeStruct((TOTAL, EMBED), jnp.float32),
    mesh=_mesh,
    scratch_types=[
        pltpu.VMEM((B_PER_W,), jnp.int32),
        pltpu.VMEM((CHUNK, EMBED), jnp.float32),
        pltpu.SemaphoreType.DMA,
    ],
    compiler_params=pltpu.CompilerParams(use_tc_tiling_on_sc=False),
)
def _gather(idx_hbm, table_hbm, out_hbm, idx_v, rows_v, sem):
    wid = lax.axis_index("s") * NC + lax.axis_index("c")
    base = wid * B_PER_W
    pltpu.sync_copy(idx_hbm.at[pl.ds(base, B_PER_W)], idx_v)

    def body(i, _):
        off = i * CHUNK
        pltpu.async_copy(
            table_hbm.at[idx_v.at[pl.ds(off, CHUNK)]], rows_v, sem
        ).wait()
        pltpu.sync_copy(rows_v, out_hbm.at[pl.ds(base + off, CHUNK)])
        return ()

    lax.fori_loop(0, N_CHUNKS, body, ())


def kernel(inputs, table):
    idx = inputs.reshape(TOTAL).astype(jnp.int32)
    out = _gather(idx, table)
    return out.reshape(B, L, EMBED)


# speedup vs baseline: 1.8750x; 1.0248x over previous
import functools

import jax
import jax.numpy as jnp
from jax import lax
from jax.experimental import pallas as pl
from jax.experimental.pallas import tpu as pltpu
from jax.experimental.pallas import tpu_sc as plsc

B = 16384
L = 50
EMBED = 64
TOTAL = B * L

_info = plsc.get_sparse_core_info()
NC = _info.num_cores
NS = _info.num_subcores
NW = NC * NS
B_PER_W = TOTAL // NW
CHUNK = 512
NBUF = 2
N_CHUNKS = B_PER_W // CHUNK
N_GROUPS = N_CHUNKS // NBUF

_mesh = plsc.VectorSubcoreMesh(core_axis_name="c", subcore_axis_name="s")


@functools.partial(
    pl.kernel,
    out_type=jax.ShapeDtypeStruct((TOTAL, EMBED), jnp.float32),
    mesh=_mesh,
    scratch_types=[
        pltpu.VMEM((B_PER_W,), jnp.int32),
        [pltpu.VMEM((CHUNK, EMBED), jnp.float32) for _ in range(NBUF)],
        [pltpu.SemaphoreType.DMA for _ in range(NBUF)],
        [pltpu.SemaphoreType.DMA for _ in range(NBUF)],
    ],
    compiler_params=pltpu.CompilerParams(use_tc_tiling_on_sc=False),
)
def _gather(idx_hbm, table_hbm, out_hbm, idx_v, rows, sem_g, sem_o):
    wid = lax.axis_index("s") * NC + lax.axis_index("c")
    base = wid * B_PER_W
    pltpu.sync_copy(idx_hbm.at[pl.ds(base, B_PER_W)], idx_v)

    def start_gather(i, b):
        pltpu.async_copy(
            table_hbm.at[idx_v.at[pl.ds(i * CHUNK, CHUNK)]], rows[b], sem_g[b]
        )

    def wait_gather(b):
        pltpu.make_async_copy(
            table_hbm.at[pl.ds(0, CHUNK)], rows[b], sem_g[b]
        ).wait()

    def start_out(i, b):
        return pltpu.async_copy(
            rows[b], out_hbm.at[pl.ds(base + i * CHUNK, CHUNK)], sem_o[b]
        )

    for b in range(NBUF):
        start_gather(b, b)

    def body(k, _):
        for b in range(NBUF):
            i = k * NBUF + b
            wait_gather(b)
            start_out(i, b).wait()
            start_gather(i + NBUF, b)
        return ()

    lax.fori_loop(0, N_GROUPS - 1, body, ())

    last = (N_GROUPS - 1) * NBUF
    outs = []
    for b in range(NBUF):
        wait_gather(b)
        outs.append(start_out(last + b, b))
    for c in outs:
        c.wait()


def kernel(inputs, table):
    idx = inputs.reshape(TOTAL).astype(jnp.int32)
    out = _gather(idx, table)
    return out.reshape(B, L, EMBED)
